# Initial kernel scaffold; baseline (speedup 1.0000x reference)
import jax
import jax.numpy as jnp
from jax.experimental import pallas as pl


def kernel(x, W0, W1, W2, W3, W4, W5, W6, W7, W8):
    raise NotImplementedError("write your pallas kernel here")


# speedup vs baseline: 2.1695x; 2.1695x over previous
import functools

import jax
import jax.numpy as jnp
from jax import lax
from jax.experimental import pallas as pl
from jax.experimental.pallas import tpu as pltpu
from jax.experimental.pallas import tpu_sc as plsc

_DIMS = [119, 5, 12, 12, 10, 6, 6, 2, 2]
_K = 8
_DOUT = 256
_N = 100000
_XW = 32
_TROWS = sum(_DIMS)

_NC = 2
_NS = 16
_NW = _NC * _NS
_RPW = _N // _NW
_CHUNK = 125
_NCHUNK = _RPW // _CHUNK

_STARTS = []
_s = 0
for _d in _DIMS:
    _STARTS.append(_s)
    _s += _d


def _sc_body(x_hbm, t_hbm, out_hbm, x_v, t_v, out_v, addr_v):
    wid = lax.axis_index("s") * _NC + lax.axis_index("c")
    pltpu.sync_copy(t_hbm, t_v)

    iota = lax.iota(jnp.int32, 16)
    off0 = iota * 0
    for i in range(8):
        off0 = jnp.where(iota == _K + i, _STARTS[i] * _DOUT, off0)
    off1 = jnp.where(iota == 0, _STARTS[8] * _DOUT, iota * 0)
    cmask = jnp.where(iota < _K, 1.0, 0.0)

    def chunk_body(c, carry):
        base = wid * _RPW + c * _CHUNK
        pltpu.sync_copy(x_hbm.at[pl.ds(base * _XW, _CHUNK * _XW)], x_v)

        def row_body(r, rcarry):
            x0 = x_v[pl.ds(r * _XW, 16)]
            x1 = x_v[pl.ds(r * _XW + 16, 16)]
            addr_v[pl.ds(0, 16)] = x0.astype(jnp.int32) * _DOUT + off0
            addr_v[pl.ds(16, 16)] = x1.astype(jnp.int32) * _DOUT + off1
            acc = [None] * 16
            for i in range(9):
                lane = 8 + i if i < 8 else 16
                b = plsc.load_gather(addr_v, [iota * 0 + lane])
                for j in range(16):
                    g = plsc.load_gather(t_v, [b + (iota + 16 * j)])
                    acc[j] = g if acc[j] is None else acc[j] + g
            acc[0] = acc[0] + x0 * cmask
            for j in range(16):
                out_v[pl.ds(r * _DOUT + 16 * j, 16)] = acc[j]
            return rcarry

        lax.fori_loop(0, _CHUNK, row_body, 0)
        pltpu.sync_copy(out_v, out_hbm.at[pl.ds(base * _DOUT, _CHUNK * _DOUT)])
        return carry

    lax.fori_loop(0, _NCHUNK, chunk_body, 0)


@jax.jit
def _run(xp_flat, t_flat):
    mesh = plsc.VectorSubcoreMesh(core_axis_name="c", subcore_axis_name="s")
    f = pl.kernel(
        _sc_body,
        mesh=mesh,
        compiler_params=pltpu.CompilerParams(needs_layout_passes=False),
        out_type=jax.ShapeDtypeStruct((_N * _DOUT,), jnp.float32),
        scratch_types=[
            pltpu.VMEM((_CHUNK * _XW,), jnp.float32),
            pltpu.VMEM((_TROWS * _DOUT,), jnp.float32),
            pltpu.VMEM((_CHUNK * _DOUT,), jnp.float32),
            pltpu.VMEM((32,), jnp.int32),
        ],
    )
    return f(xp_flat, t_flat)


def kernel(x, W0, W1, W2, W3, W4, W5, W6, W7, W8):
    Ws = [W0, W1, W2, W3, W4, W5, W6, W7, W8]
    table = jnp.concatenate(Ws, axis=0)
    table = jnp.pad(table, ((0, 0), (_K, 0)))
    xp = jnp.pad(x, ((0, 0), (0, _XW - x.shape[1])))
    out = _run(xp.reshape(-1), table.reshape(-1))
    return out.reshape(_N, _DOUT)
